# baseline (speedup 1.0000x reference)
import functools

import jax
import jax.numpy as jnp
from jax import lax
from jax.experimental import pallas as pl
from jax.experimental.pallas import tpu as pltpu
from jax.experimental.pallas import tpu_sc as plsc

_INFO = plsc.get_sparse_core_info()
_NC = _INFO.num_cores
_NS = _INFO.num_subcores
_NW = _NC * _NS
_CHUNK = 128


@functools.lru_cache(maxsize=None)
def _make_lookup(V, D, B):
    b_per_w = B // _NW
    n_chunks = b_per_w // _CHUNK
    mesh = plsc.VectorSubcoreMesh(core_axis_name="c", subcore_axis_name="s")

    @functools.partial(
        pl.kernel,
        mesh=mesh,
        out_type=jax.ShapeDtypeStruct((B * D,), jnp.float32),
        scratch_types=[
            pltpu.VMEM((b_per_w,), jnp.int32),
            pltpu.VMEM((n_chunks, _CHUNK), jnp.int32),
            pltpu.VMEM((b_per_w, 2 * D), jnp.float32),
            pltpu.VMEM((b_per_w * D,), jnp.float32),
            pltpu.SemaphoreType.DMA,
        ],
    )
    def lookup(packed_hbm, idx_hbm, out_hbm, idx_v, p_v, pairs_v, rows_v, sem):
        wid = lax.axis_index("s") * _NC + lax.axis_index("c")
        base = wid * b_per_w
        pltpu.sync_copy(idx_hbm.at[pl.ds(base, b_per_w)], idx_v)

        for r in range(n_chunks):

            def mkp(g, _, r=r):
                vec = idx_v[pl.ds(r * _CHUNK + g * 16, 16)]
                p_v[r, pl.ds(g * 16, 16)] = vec >> 1
                return 0

            lax.fori_loop(0, _CHUNK // 16, mkp, 0)

        copies = [
            pltpu.async_copy(
                packed_hbm.at[p_v.at[r]],
                pairs_v.at[pl.ds(r * _CHUNK, _CHUNK)],
                sem,
            )
            for r in range(n_chunks)
        ]
        for c in copies:
            c.wait()

        def sel(g, _):
            vec = idx_v[pl.ds(g * 16, 16)]
            for j in range(16):
                k = g * 16 + j
                half = (vec[j] & 1) * D
                for t in range(D // 16):
                    rows_v[pl.ds(k * D + t * 16, 16)] = pairs_v[
                        k, pl.ds(half + t * 16, 16)
                    ]
            return 0

        lax.fori_loop(0, b_per_w // 16, sel, 0)
        pltpu.sync_copy(rows_v, out_hbm.at[pl.ds(base * D, b_per_w * D)])

    return lookup


def kernel(x, age_embedding_weight):
    (B,) = x.shape
    V, D = age_embedding_weight.shape
    packed = age_embedding_weight.reshape(V // 2, 2 * D)
    flat = _make_lookup(V, D, B)(packed, x.astype(jnp.int32))
    return flat.reshape(B, D)
